# baseline (speedup 1.0000x reference)
import functools

import jax
import jax.numpy as jnp
from jax import lax
from jax.experimental import pallas as pl
from jax.experimental.pallas import tpu as pltpu
from jax.experimental.pallas import tpu_sc as plsc

N = 10000
E = 160000
F = 16
D = 128
B = 50
L = 200
NT = 8
ET = 16
DEPTH = 3

NC = 2
NS = 16
NW = NC * NS

NP = 10240
EP = 163840
BP = 56
KE = 128
NCHUNK = EP // (NW * KE)
EW = NCHUNK * KE
ROWS_W = NP // NW
KG = 80
GCHUNK = ROWS_W // KG


def _pre_body(nf_ref, nt_ref, wf_ref, bf_ref, te_ref, ee_ref, wtop_ref,
              wbot_ref, bm_ref, pre_ref, npw_ref, eb_ref):
    dense = jnp.tanh(
        jnp.dot(nf_ref[...], wf_ref[...], preferred_element_type=jnp.float32)
        + bf_ref[...])
    oh = (nt_ref[...] == lax.broadcasted_iota(jnp.int32, (N, NT), 1)
          ).astype(jnp.float32)
    pre = dense + jnp.dot(oh, te_ref[...], preferred_element_type=jnp.float32, precision=lax.Precision.HIGHEST)
    pre_ref[...] = pre
    npw_ref[...] = jnp.dot(pre, wtop_ref[...],
                           preferred_element_type=jnp.float32)
    eb_ref[...] = jnp.dot(ee_ref[...], wbot_ref[...],
                          preferred_element_type=jnp.float32) + bm_ref[...]


def _pre_call(nf, nt2, w_feat, b_feat, type_emb, edge_emb, wtop, wbot, bm):
    return pl.pallas_call(
        _pre_body,
        out_shape=(
            jax.ShapeDtypeStruct((N, D), jnp.float32),
            jax.ShapeDtypeStruct((N, D), jnp.float32),
            jax.ShapeDtypeStruct((ET, D), jnp.float32),
        ),
    )(nf, nt2, w_feat, b_feat, type_emb, edge_emb, wtop, wbot, bm)


def _gru_body(x0_ref, wih_ref, whh_ref, bih_ref, bhh_ref, out_ref,
              seq_ref, h_ref):
    l = pl.program_id(0)
    t = pl.program_id(1)

    @pl.when(t == 0)
    def _():
        h_ref[...] = jnp.zeros_like(h_ref)

    x_seq = seq_ref[pl.ds(t, 1)][0]
    xt = jnp.where(l == 0, x0_ref[0], x_seq)
    h = h_ref[...]
    gi = jnp.dot(xt, wih_ref[0], preferred_element_type=jnp.float32) \
        + bih_ref[0]
    gh = jnp.dot(h, whh_ref[0], preferred_element_type=jnp.float32) \
        + bhh_ref[0]
    r = jax.nn.sigmoid(gi[:, :D] + gh[:, :D])
    z = jax.nn.sigmoid(gi[:, D:2 * D] + gh[:, D:2 * D])
    n = jnp.tanh(gi[:, 2 * D:] + r * gh[:, 2 * D:])
    hnew = (1.0 - z) * n + z * h
    h_ref[...] = hnew
    seq_ref[pl.ds(t, 1)] = hnew[None]
    out_ref[0] = hnew


def _gru_call(x0p, wih_all, whh_all, bih_all, bhh_all):
    return pl.pallas_call(
        _gru_body,
        grid=(3, L),
        in_specs=[
            pl.BlockSpec((1, BP, D), lambda l, t: (t, 0, 0)),
            pl.BlockSpec((1, D, 3 * D), lambda l, t: (l, 0, 0)),
            pl.BlockSpec((1, D, 3 * D), lambda l, t: (l, 0, 0)),
            pl.BlockSpec((1, 1, 3 * D), lambda l, t: (l, 0, 0)),
            pl.BlockSpec((1, 1, 3 * D), lambda l, t: (l, 0, 0)),
        ],
        out_specs=pl.BlockSpec((1, BP, D), lambda l, t: (t, 0, 0)),
        out_shape=jax.ShapeDtypeStruct((L, BP, D), jnp.float32),
        scratch_shapes=[
            pltpu.VMEM((L, BP, D), jnp.float32),
            pltpu.VMEM((BP, D), jnp.float32),
        ],
    )(x0p, wih_all, whh_all, bih_all, bhh_all)


def _gather2_body(pre_hbm, npw_hbm, idx_hbm, h0_hbm, hw0_hbm,
                  idx_v, rows_a, rows_b, sem_a, sem_b):
    c = lax.axis_index("c")
    s = lax.axis_index("s")
    w = s * NC + c
    base = w * ROWS_W

    def chunk(i, _):
        off = base + i * KG
        pltpu.sync_copy(idx_hbm.at[pl.ds(off, KG)], idx_v)
        cp_a = pltpu.async_copy(pre_hbm.at[idx_v], rows_a, sem_a)
        cp_b = pltpu.async_copy(npw_hbm.at[idx_v], rows_b, sem_b)
        cp_a.wait()
        cp_b.wait()
        pltpu.sync_copy(rows_a, h0_hbm.at[pl.ds(off, KG)])
        pltpu.sync_copy(rows_b, hw0_hbm.at[pl.ds(off, KG)])
        return 0

    lax.fori_loop(0, GCHUNK, chunk, 0)


def _gather2_call(node_pre, npw, idxp):
    mesh = plsc.VectorSubcoreMesh(core_axis_name="c", subcore_axis_name="s")
    f = pl.kernel(
        _gather2_body,
        out_type=(
            jax.ShapeDtypeStruct((NP, D), jnp.float32),
            jax.ShapeDtypeStruct((NP, D), jnp.float32),
        ),
        mesh=mesh,
        scratch_types=[
            pltpu.VMEM((KG,), jnp.int32),
            pltpu.VMEM((KG, D), jnp.float32),
            pltpu.VMEM((KG, D), jnp.float32),
            pltpu.SemaphoreType.DMA,
            pltpu.SemaphoreType.DMA,
        ],
    )
    return f(node_pre, npw, idxp)


NCH0 = 40
NCH1 = (EP // KE - NS * NCH0) // NS


def _mp_body(tab_hbm, cmb_hbm, dst_hbm, zero_hbm, parts_hbm,
             acc_sh, cmb_v, dst_v, rows_a, rows_b,
             sem_ga, sem_gb, sem_sa, sem_sb):
    c = lax.axis_index("c")
    s = lax.axis_index("s")
    stripe = NP // NS

    def run(nch, rowbase):
        pltpu.sync_copy(cmb_hbm.at[pl.ds(rowbase, nch)],
                        cmb_v.at[pl.ds(0, nch)])
        pltpu.sync_copy(dst_hbm.at[pl.ds(rowbase, nch)],
                        dst_v.at[pl.ds(0, nch)])

        pltpu.sync_copy(zero_hbm.at[pl.ds(s * stripe, stripe)],
                        acc_sh.at[pl.ds(s * stripe, stripe)])
        plsc.subcore_barrier()

        def pair(i, _):
            j = 2 * i
            ga = pltpu.async_copy(tab_hbm.at[cmb_v.at[j]], rows_a, sem_ga)
            gb = pltpu.async_copy(tab_hbm.at[cmb_v.at[j + 1]], rows_b,
                                  sem_gb)
            ga.wait()
            sa = pltpu.async_copy(rows_a, acc_sh.at[dst_v.at[j]], sem_sa,
                                  add=True)
            gb.wait()
            sb = pltpu.async_copy(rows_b, acc_sh.at[dst_v.at[j + 1]],
                                  sem_sb, add=True)
            sa.wait()
            sb.wait()
            return 0

        lax.fori_loop(0, nch // 2, pair, 0)

    @pl.when(c == 0)
    def _():
        run(NCH0, s * NCH0)

    @pl.when(c == 1)
    def _():
        run(NCH1, NS * NCH0 + s * NCH1)

    plsc.subcore_barrier()
    pltpu.sync_copy(acc_sh.at[pl.ds(s * stripe, stripe)],
                    parts_hbm.at[c, pl.ds(s * stripe, stripe)])


def _mp_call(tab, cmb3, dst3, zeros_np):
    mesh = plsc.VectorSubcoreMesh(core_axis_name="c", subcore_axis_name="s")
    f = pl.kernel(
        _mp_body,
        out_type=jax.ShapeDtypeStruct((NC, NP, D), jnp.float32),
        mesh=mesh,
        scratch_types=[
            pltpu.VMEM_SHARED((NP, D), jnp.float32),
            pltpu.VMEM((NCH0, KE), jnp.int32),
            pltpu.VMEM((NCH0, KE), jnp.int32),
            pltpu.VMEM((KE, D), jnp.float32),
            pltpu.VMEM((KE, D), jnp.float32),
            pltpu.SemaphoreType.DMA,
            pltpu.SemaphoreType.DMA,
            pltpu.SemaphoreType.DMA,
            pltpu.SemaphoreType.DMA,
        ],
    )
    return f(tab, cmb3, dst3, zeros_np)


_TRB = 1000


def _table_body(hw_ref, eb_ref, tab_ref):
    t = jnp.tanh(hw_ref[...][:, None, :] + eb_ref[...][None, :, :])
    tab_ref[...] = t.reshape(_TRB * ET, D)


def _table_call(hw, ebias):
    return pl.pallas_call(
        _table_body,
        grid=(N // _TRB,),
        in_specs=[
            pl.BlockSpec((_TRB, D), lambda i: (i, 0)),
            pl.BlockSpec((ET, D), lambda i: (0, 0)),
        ],
        out_specs=pl.BlockSpec((_TRB * ET, D), lambda i: (i, 0)),
        out_shape=jax.ShapeDtypeStruct((N * ET, D), jnp.float32),
    )(hw, ebias)


def _cell_body(p0_ref, p1_ref, h_ref, wih_ref, whh_ref, bih_ref, bhh_ref,
               hn_ref):
    agg = p0_ref[...] + p1_ref[...]
    h = h_ref[...]
    gi = jnp.dot(agg, wih_ref[...], preferred_element_type=jnp.float32) \
        + bih_ref[...]
    gh = jnp.dot(h, whh_ref[...], preferred_element_type=jnp.float32) \
        + bhh_ref[...]
    r = jax.nn.sigmoid(gi[:, :D] + gh[:, :D])
    z = jax.nn.sigmoid(gi[:, D:2 * D] + gh[:, D:2 * D])
    n = jnp.tanh(gi[:, 2 * D:] + r * gh[:, 2 * D:])
    hn_ref[...] = (1.0 - z) * n + z * h


def _cellt_body(p0_ref, p1_ref, h_ref, wih_ref, whh_ref, bih_ref, bhh_ref,
                wtop_ref, eb_ref, hn_ref, tab_ref):
    agg = p0_ref[...] + p1_ref[...]
    h = h_ref[...]
    gi = jnp.dot(agg, wih_ref[...], preferred_element_type=jnp.float32) \
        + bih_ref[...]
    gh = jnp.dot(h, whh_ref[...], preferred_element_type=jnp.float32) \
        + bhh_ref[...]
    r = jax.nn.sigmoid(gi[:, :D] + gh[:, :D])
    z = jax.nn.sigmoid(gi[:, D:2 * D] + gh[:, D:2 * D])
    n = jnp.tanh(gi[:, 2 * D:] + r * gh[:, 2 * D:])
    hnew = (1.0 - z) * n + z * h
    hn_ref[...] = hnew
    hw = jnp.dot(hnew, wtop_ref[...], preferred_element_type=jnp.float32)
    t = jnp.tanh(hw[:, None, :] + eb_ref[...][None, :, :])
    tab_ref[...] = t.reshape(_RB * ET, D)


_RB = 1000


def _cell_call(parts, h, wih, whh, bih, bhh):
    p0 = parts[0, :N]
    p1 = parts[1, :N]
    return pl.pallas_call(
        _cell_body,
        grid=(N // _RB,),
        in_specs=[
            pl.BlockSpec((_RB, D), lambda i: (i, 0)),
            pl.BlockSpec((_RB, D), lambda i: (i, 0)),
            pl.BlockSpec((_RB, D), lambda i: (i, 0)),
            pl.BlockSpec((D, 3 * D), lambda i: (0, 0)),
            pl.BlockSpec((D, 3 * D), lambda i: (0, 0)),
            pl.BlockSpec((1, 3 * D), lambda i: (0, 0)),
            pl.BlockSpec((1, 3 * D), lambda i: (0, 0)),
        ],
        out_specs=pl.BlockSpec((_RB, D), lambda i: (i, 0)),
        out_shape=jax.ShapeDtypeStruct((N, D), jnp.float32),
    )(p0, p1, h, wih, whh, bih, bhh)


def _cellt_call(parts, h, wih, whh, bih, bhh, wtop, ebias):
    p0 = parts[0, :N]
    p1 = parts[1, :N]
    return pl.pallas_call(
        _cellt_body,
        grid=(N // _RB,),
        in_specs=[
            pl.BlockSpec((_RB, D), lambda i: (i, 0)),
            pl.BlockSpec((_RB, D), lambda i: (i, 0)),
            pl.BlockSpec((_RB, D), lambda i: (i, 0)),
            pl.BlockSpec((D, 3 * D), lambda i: (0, 0)),
            pl.BlockSpec((D, 3 * D), lambda i: (0, 0)),
            pl.BlockSpec((1, 3 * D), lambda i: (0, 0)),
            pl.BlockSpec((1, 3 * D), lambda i: (0, 0)),
            pl.BlockSpec((D, D), lambda i: (0, 0)),
            pl.BlockSpec((ET, D), lambda i: (0, 0)),
        ],
        out_specs=(
            pl.BlockSpec((_RB, D), lambda i: (i, 0)),
            pl.BlockSpec((_RB * ET, D), lambda i: (i, 0)),
        ),
        out_shape=(
            jax.ShapeDtypeStruct((N, D), jnp.float32),
            jax.ShapeDtypeStruct((N * ET, D), jnp.float32),
        ),
    )(p0, p1, h, wih, whh, bih, bhh, wtop, ebias)


def _post_body(np_ref, go_ref, gid_ref, lgni_ref, wp_ref, bp_ref,
               wm_ref, bm_ref, out_ref):
    ohg = (lax.broadcasted_iota(jnp.int32, (B, N), 0) == gid_ref[...]
           ).astype(jnp.float32)
    pooled = jnp.dot(ohg, np_ref[...], preferred_element_type=jnp.float32, precision=lax.Precision.HIGHEST)
    gp = jnp.tanh(
        jnp.dot(pooled, wp_ref[...], preferred_element_type=jnp.float32)
        + bp_ref[...])
    ohl = (lax.broadcasted_iota(jnp.int32, (B, N), 1) == lgni_ref[...]
           ).astype(jnp.float32)
    ge = jnp.dot(ohl, go_ref[...], preferred_element_type=jnp.float32, precision=lax.Precision.HIGHEST)
    cat = jnp.concatenate([ge, gp], axis=-1)
    out_ref[...] = (
        jnp.dot(cat, wm_ref[...], preferred_element_type=jnp.float32)
        + bm_ref[...])


def _post_call(node_post, gru_out, gid2, lgni2, w_post, b_post,
               w_merge, b_merge):
    return pl.pallas_call(
        _post_body,
        out_shape=jax.ShapeDtypeStruct((B, D), jnp.float32),
    )(node_post, gru_out, gid2, lgni2, w_post, b_post, w_merge, b_merge)


def kernel(node_features, node_types, last_graph_node_index,
           node_features_graph_index, edge_types, edge_index, graph_ids,
           W_feat, b_feat, type_emb, gru_params, edge_emb, W_msg, b_msg,
           cell_Wih, cell_Whh, cell_bih, cell_bhh, W_post, b_post,
           W_merge, b_merge):
    nt2 = node_types.astype(jnp.int32).reshape(N, 1)
    wtop = W_msg[:D]
    wbot = W_msg[D:]
    bm2 = b_msg.reshape(1, D)

    node_pre, npw, ebias = _pre_call(
        node_features, nt2, W_feat, b_feat.reshape(1, D), type_emb,
        edge_emb, wtop, wbot, bm2)

    x0p = jnp.pad(node_pre.reshape(L, B, D), ((0, 0), (0, BP - B), (0, 0)))
    wih_all = jnp.stack([g[0] for g in gru_params])
    whh_all = jnp.stack([g[1] for g in gru_params])
    bih_all = jnp.stack([g[2] for g in gru_params]).reshape(3, 1, 3 * D)
    bhh_all = jnp.stack([g[3] for g in gru_params]).reshape(3, 1, 3 * D)
    gru_seq = _gru_call(x0p, wih_all, whh_all, bih_all, bhh_all)
    gru_out = gru_seq[:, :B, :].reshape(N, D)

    idxp = jnp.pad(node_features_graph_index.astype(jnp.int32), (0, NP - N))
    h0p, hw0p = _gather2_call(node_pre, npw, idxp)
    h = h0p[:N]
    tab = _table_call(hw0p[:N], ebias)

    srcp = jnp.pad(edge_index[0].astype(jnp.int32), (0, EP - E))
    etp = jnp.pad(edge_types.astype(jnp.int32), (0, EP - E))
    dstp = jnp.pad(edge_index[1].astype(jnp.int32), (0, EP - E),
                   constant_values=N)
    cmb3 = (srcp * ET + etp).reshape(EP // KE, KE)
    dst3 = dstp.reshape(EP // KE, KE)
    zeros_np = jnp.zeros((NP, D), jnp.float32)

    for d in range(DEPTH):
        parts = _mp_call(tab, cmb3, dst3, zeros_np)
        if d < DEPTH - 1:
            h, tab = _cellt_call(parts, h, cell_Wih, cell_Whh,
                                 cell_bih.reshape(1, 3 * D),
                                 cell_bhh.reshape(1, 3 * D), wtop, ebias)
        else:
            h = _cell_call(parts, h, cell_Wih, cell_Whh,
                           cell_bih.reshape(1, 3 * D),
                           cell_bhh.reshape(1, 3 * D))

    node_post = h
    merged = _post_call(
        node_post, gru_out,
        graph_ids.astype(jnp.int32).reshape(1, N),
        last_graph_node_index.astype(jnp.int32).reshape(B, 1),
        W_post, b_post.reshape(1, 2 * D),
        W_merge, b_merge.reshape(1, D))
    return (node_post, merged)


# speedup vs baseline: 1.0418x; 1.0418x over previous
import functools

import jax
import jax.numpy as jnp
from jax import lax
from jax.experimental import pallas as pl
from jax.experimental.pallas import tpu as pltpu
from jax.experimental.pallas import tpu_sc as plsc

N = 10000
E = 160000
F = 16
D = 128
B = 50
L = 200
NT = 8
ET = 16
DEPTH = 3

NC = 2
NS = 16
NW = NC * NS

NP = 10240
EP = 163840
BP = 56
KE = 128
NCHUNK = EP // (NW * KE)
EW = NCHUNK * KE
ROWS_W = NP // NW
KG = 80
GCHUNK = ROWS_W // KG


def _pre_body(nf_ref, nt_ref, wf_ref, bf_ref, te_ref, ee_ref, wtop_ref,
              wbot_ref, bm_ref, pre_ref, npw_ref, eb_ref):
    dense = jnp.tanh(
        jnp.dot(nf_ref[...], wf_ref[...], preferred_element_type=jnp.float32)
        + bf_ref[...])
    oh = (nt_ref[...] == lax.broadcasted_iota(jnp.int32, (N, NT), 1)
          ).astype(jnp.float32)
    pre = dense + jnp.dot(oh, te_ref[...], preferred_element_type=jnp.float32, precision=lax.Precision.HIGHEST)
    pre_ref[...] = pre
    npw_ref[...] = jnp.dot(pre, wtop_ref[...],
                           preferred_element_type=jnp.float32)
    eb_ref[...] = jnp.dot(ee_ref[...], wbot_ref[...],
                          preferred_element_type=jnp.float32) + bm_ref[...]


def _pre_call(nf, nt2, w_feat, b_feat, type_emb, edge_emb, wtop, wbot, bm):
    return pl.pallas_call(
        _pre_body,
        out_shape=(
            jax.ShapeDtypeStruct((N, D), jnp.float32),
            jax.ShapeDtypeStruct((N, D), jnp.float32),
            jax.ShapeDtypeStruct((ET, D), jnp.float32),
        ),
    )(nf, nt2, w_feat, b_feat, type_emb, edge_emb, wtop, wbot, bm)


def _gru_body(x0_ref, wih_ref, whh_ref, bih_ref, bhh_ref, out_ref,
              seq_ref, h_ref, gi_ref):
    l = pl.program_id(0)
    t = pl.program_id(1)

    @pl.when(t == 0)
    def _():
        h_ref[...] = jnp.zeros_like(h_ref)
        x_all = jnp.where(l == 0, x0_ref[...], seq_ref[...])
        gi_all = jnp.dot(x_all.reshape(L * BP, D), wih_ref[0],
                         preferred_element_type=jnp.float32) + bih_ref[0]
        gi_ref[...] = gi_all.reshape(L, BP, 3 * D)

    gi = gi_ref[pl.ds(t, 1)][0]
    h = h_ref[...]
    gh = jnp.dot(h, whh_ref[0], preferred_element_type=jnp.float32) \
        + bhh_ref[0]
    r = jax.nn.sigmoid(gi[:, :D] + gh[:, :D])
    z = jax.nn.sigmoid(gi[:, D:2 * D] + gh[:, D:2 * D])
    n = jnp.tanh(gi[:, 2 * D:] + r * gh[:, 2 * D:])
    hnew = (1.0 - z) * n + z * h
    h_ref[...] = hnew
    seq_ref[pl.ds(t, 1)] = hnew[None]
    out_ref[0] = hnew


def _gru_call(x0p, wih_all, whh_all, bih_all, bhh_all):
    return pl.pallas_call(
        _gru_body,
        grid=(3, L),
        in_specs=[
            pl.BlockSpec((L, BP, D), lambda l, t: (0, 0, 0)),
            pl.BlockSpec((1, D, 3 * D), lambda l, t: (l, 0, 0)),
            pl.BlockSpec((1, D, 3 * D), lambda l, t: (l, 0, 0)),
            pl.BlockSpec((1, 1, 3 * D), lambda l, t: (l, 0, 0)),
            pl.BlockSpec((1, 1, 3 * D), lambda l, t: (l, 0, 0)),
        ],
        out_specs=pl.BlockSpec((1, BP, D), lambda l, t: (t, 0, 0)),
        out_shape=jax.ShapeDtypeStruct((L, BP, D), jnp.float32),
        scratch_shapes=[
            pltpu.VMEM((L, BP, D), jnp.float32),
            pltpu.VMEM((BP, D), jnp.float32),
            pltpu.VMEM((L, BP, 3 * D), jnp.float32),
        ],
    )(x0p, wih_all, whh_all, bih_all, bhh_all)


def _gather2_body(pre_hbm, npw_hbm, idx_hbm, h0_hbm, hw0_hbm,
                  idx_v, rows_a, rows_b, sem_a, sem_b):
    c = lax.axis_index("c")
    s = lax.axis_index("s")
    w = s * NC + c
    base = w * ROWS_W

    def chunk(i, _):
        off = base + i * KG
        pltpu.sync_copy(idx_hbm.at[pl.ds(off, KG)], idx_v)
        cp_a = pltpu.async_copy(pre_hbm.at[idx_v], rows_a, sem_a)
        cp_b = pltpu.async_copy(npw_hbm.at[idx_v], rows_b, sem_b)
        cp_a.wait()
        cp_b.wait()
        pltpu.sync_copy(rows_a, h0_hbm.at[pl.ds(off, KG)])
        pltpu.sync_copy(rows_b, hw0_hbm.at[pl.ds(off, KG)])
        return 0

    lax.fori_loop(0, GCHUNK, chunk, 0)


def _gather2_call(node_pre, npw, idxp):
    mesh = plsc.VectorSubcoreMesh(core_axis_name="c", subcore_axis_name="s")
    f = pl.kernel(
        _gather2_body,
        out_type=(
            jax.ShapeDtypeStruct((NP, D), jnp.float32),
            jax.ShapeDtypeStruct((NP, D), jnp.float32),
        ),
        mesh=mesh,
        scratch_types=[
            pltpu.VMEM((KG,), jnp.int32),
            pltpu.VMEM((KG, D), jnp.float32),
            pltpu.VMEM((KG, D), jnp.float32),
            pltpu.SemaphoreType.DMA,
            pltpu.SemaphoreType.DMA,
        ],
    )
    return f(node_pre, npw, idxp)


NCH0 = 40
NCH1 = (EP // KE - NS * NCH0) // NS


def _mp_body(tab_hbm, cmb_hbm, dst_hbm, zero_hbm, parts_hbm,
             acc_sh, cmb_v, dst_v, rows_a, sem_ga):
    c = lax.axis_index("c")
    s = lax.axis_index("s")
    stripe = NP // NS

    def run(nch, rowbase):
        pltpu.sync_copy(cmb_hbm.at[pl.ds(rowbase, nch)],
                        cmb_v.at[pl.ds(0, nch)])
        pltpu.sync_copy(dst_hbm.at[pl.ds(rowbase, nch)],
                        dst_v.at[pl.ds(0, nch)])

        pltpu.sync_copy(zero_hbm.at[pl.ds(s * stripe, stripe)],
                        acc_sh.at[pl.ds(s * stripe, stripe)])
        plsc.subcore_barrier()

        def chunk(i, _):
            cp = pltpu.async_copy(tab_hbm.at[cmb_v.at[i]], rows_a, sem_ga)
            cp.wait()
            pltpu.sync_copy(rows_a, acc_sh.at[dst_v.at[i]], add=True)
            return 0

        lax.fori_loop(0, nch, chunk, 0)

    @pl.when(c == 0)
    def _():
        run(NCH0, s * NCH0)

    @pl.when(c == 1)
    def _():
        run(NCH1, NS * NCH0 + s * NCH1)

    plsc.subcore_barrier()
    pltpu.sync_copy(acc_sh.at[pl.ds(s * stripe, stripe)],
                    parts_hbm.at[c, pl.ds(s * stripe, stripe)])


def _mp_call(tab, cmb3, dst3, zeros_np):
    mesh = plsc.VectorSubcoreMesh(core_axis_name="c", subcore_axis_name="s")
    f = pl.kernel(
        _mp_body,
        out_type=jax.ShapeDtypeStruct((NC, NP, D), jnp.float32),
        mesh=mesh,
        scratch_types=[
            pltpu.VMEM_SHARED((NP, D), jnp.float32),
            pltpu.VMEM((NCH0, KE), jnp.int32),
            pltpu.VMEM((NCH0, KE), jnp.int32),
            pltpu.VMEM((KE, D), jnp.float32),
            pltpu.SemaphoreType.DMA,
        ],
    )
    return f(tab, cmb3, dst3, zeros_np)


_TRB = 1000


def _table_body(hw_ref, eb_ref, tab_ref):
    t = jnp.tanh(hw_ref[...][:, None, :] + eb_ref[...][None, :, :])
    tab_ref[...] = t.reshape(_TRB * ET, D)


def _table_call(hw, ebias):
    return pl.pallas_call(
        _table_body,
        grid=(N // _TRB,),
        in_specs=[
            pl.BlockSpec((_TRB, D), lambda i: (i, 0)),
            pl.BlockSpec((ET, D), lambda i: (0, 0)),
        ],
        out_specs=pl.BlockSpec((_TRB * ET, D), lambda i: (i, 0)),
        out_shape=jax.ShapeDtypeStruct((N * ET, D), jnp.float32),
    )(hw, ebias)


def _cell_body(p0_ref, p1_ref, h_ref, wih_ref, whh_ref, bih_ref, bhh_ref,
               hn_ref):
    agg = p0_ref[...] + p1_ref[...]
    h = h_ref[...]
    gi = jnp.dot(agg, wih_ref[...], preferred_element_type=jnp.float32) \
        + bih_ref[...]
    gh = jnp.dot(h, whh_ref[...], preferred_element_type=jnp.float32) \
        + bhh_ref[...]
    r = jax.nn.sigmoid(gi[:, :D] + gh[:, :D])
    z = jax.nn.sigmoid(gi[:, D:2 * D] + gh[:, D:2 * D])
    n = jnp.tanh(gi[:, 2 * D:] + r * gh[:, 2 * D:])
    hn_ref[...] = (1.0 - z) * n + z * h


def _cellt_body(p0_ref, p1_ref, h_ref, wih_ref, whh_ref, bih_ref, bhh_ref,
                wtop_ref, eb_ref, hn_ref, tab_ref):
    agg = p0_ref[...] + p1_ref[...]
    h = h_ref[...]
    gi = jnp.dot(agg, wih_ref[...], preferred_element_type=jnp.float32) \
        + bih_ref[...]
    gh = jnp.dot(h, whh_ref[...], preferred_element_type=jnp.float32) \
        + bhh_ref[...]
    r = jax.nn.sigmoid(gi[:, :D] + gh[:, :D])
    z = jax.nn.sigmoid(gi[:, D:2 * D] + gh[:, D:2 * D])
    n = jnp.tanh(gi[:, 2 * D:] + r * gh[:, 2 * D:])
    hnew = (1.0 - z) * n + z * h
    hn_ref[...] = hnew
    hw = jnp.dot(hnew, wtop_ref[...], preferred_element_type=jnp.float32)
    t = jnp.tanh(hw[:, None, :] + eb_ref[...][None, :, :])
    tab_ref[...] = t.reshape(_RB * ET, D)


_RB = 1000


def _cell_call(parts, h, wih, whh, bih, bhh):
    p0 = parts[0, :N]
    p1 = parts[1, :N]
    return pl.pallas_call(
        _cell_body,
        grid=(N // _RB,),
        in_specs=[
            pl.BlockSpec((_RB, D), lambda i: (i, 0)),
            pl.BlockSpec((_RB, D), lambda i: (i, 0)),
            pl.BlockSpec((_RB, D), lambda i: (i, 0)),
            pl.BlockSpec((D, 3 * D), lambda i: (0, 0)),
            pl.BlockSpec((D, 3 * D), lambda i: (0, 0)),
            pl.BlockSpec((1, 3 * D), lambda i: (0, 0)),
            pl.BlockSpec((1, 3 * D), lambda i: (0, 0)),
        ],
        out_specs=pl.BlockSpec((_RB, D), lambda i: (i, 0)),
        out_shape=jax.ShapeDtypeStruct((N, D), jnp.float32),
    )(p0, p1, h, wih, whh, bih, bhh)


def _cellt_call(parts, h, wih, whh, bih, bhh, wtop, ebias):
    p0 = parts[0, :N]
    p1 = parts[1, :N]
    return pl.pallas_call(
        _cellt_body,
        grid=(N // _RB,),
        in_specs=[
            pl.BlockSpec((_RB, D), lambda i: (i, 0)),
            pl.BlockSpec((_RB, D), lambda i: (i, 0)),
            pl.BlockSpec((_RB, D), lambda i: (i, 0)),
            pl.BlockSpec((D, 3 * D), lambda i: (0, 0)),
            pl.BlockSpec((D, 3 * D), lambda i: (0, 0)),
            pl.BlockSpec((1, 3 * D), lambda i: (0, 0)),
            pl.BlockSpec((1, 3 * D), lambda i: (0, 0)),
            pl.BlockSpec((D, D), lambda i: (0, 0)),
            pl.BlockSpec((ET, D), lambda i: (0, 0)),
        ],
        out_specs=(
            pl.BlockSpec((_RB, D), lambda i: (i, 0)),
            pl.BlockSpec((_RB * ET, D), lambda i: (i, 0)),
        ),
        out_shape=(
            jax.ShapeDtypeStruct((N, D), jnp.float32),
            jax.ShapeDtypeStruct((N * ET, D), jnp.float32),
        ),
    )(p0, p1, h, wih, whh, bih, bhh, wtop, ebias)


def _post_body(np_ref, go_ref, gid_ref, lgni_ref, wp_ref, bp_ref,
               wm_ref, bm_ref, out_ref):
    ohg = (lax.broadcasted_iota(jnp.int32, (B, N), 0) == gid_ref[...]
           ).astype(jnp.float32)
    pooled = jnp.dot(ohg, np_ref[...], preferred_element_type=jnp.float32, precision=lax.Precision.HIGHEST)
    gp = jnp.tanh(
        jnp.dot(pooled, wp_ref[...], preferred_element_type=jnp.float32)
        + bp_ref[...])
    ohl = (lax.broadcasted_iota(jnp.int32, (B, N), 1) == lgni_ref[...]
           ).astype(jnp.float32)
    ge = jnp.dot(ohl, go_ref[...], preferred_element_type=jnp.float32, precision=lax.Precision.HIGHEST)
    cat = jnp.concatenate([ge, gp], axis=-1)
    out_ref[...] = (
        jnp.dot(cat, wm_ref[...], preferred_element_type=jnp.float32)
        + bm_ref[...])


def _post_call(node_post, gru_out, gid2, lgni2, w_post, b_post,
               w_merge, b_merge):
    return pl.pallas_call(
        _post_body,
        out_shape=jax.ShapeDtypeStruct((B, D), jnp.float32),
    )(node_post, gru_out, gid2, lgni2, w_post, b_post, w_merge, b_merge)


def kernel(node_features, node_types, last_graph_node_index,
           node_features_graph_index, edge_types, edge_index, graph_ids,
           W_feat, b_feat, type_emb, gru_params, edge_emb, W_msg, b_msg,
           cell_Wih, cell_Whh, cell_bih, cell_bhh, W_post, b_post,
           W_merge, b_merge):
    nt2 = node_types.astype(jnp.int32).reshape(N, 1)
    wtop = W_msg[:D]
    wbot = W_msg[D:]
    bm2 = b_msg.reshape(1, D)

    node_pre, npw, ebias = _pre_call(
        node_features, nt2, W_feat, b_feat.reshape(1, D), type_emb,
        edge_emb, wtop, wbot, bm2)

    x0p = jnp.pad(node_pre.reshape(L, B, D), ((0, 0), (0, BP - B), (0, 0)))
    wih_all = jnp.stack([g[0] for g in gru_params])
    whh_all = jnp.stack([g[1] for g in gru_params])
    bih_all = jnp.stack([g[2] for g in gru_params]).reshape(3, 1, 3 * D)
    bhh_all = jnp.stack([g[3] for g in gru_params]).reshape(3, 1, 3 * D)
    gru_seq = _gru_call(x0p, wih_all, whh_all, bih_all, bhh_all)
    gru_out = gru_seq[:, :B, :].reshape(N, D)

    idxp = jnp.pad(node_features_graph_index.astype(jnp.int32), (0, NP - N))
    h0p, hw0p = _gather2_call(node_pre, npw, idxp)
    h = h0p[:N]
    tab = _table_call(hw0p[:N], ebias)

    srcp = jnp.pad(edge_index[0].astype(jnp.int32), (0, EP - E))
    etp = jnp.pad(edge_types.astype(jnp.int32), (0, EP - E))
    dstp = jnp.pad(edge_index[1].astype(jnp.int32), (0, EP - E),
                   constant_values=N)
    cmb3 = (srcp * ET + etp).reshape(EP // KE, KE)
    dst3 = dstp.reshape(EP // KE, KE)
    zeros_np = jnp.zeros((NP, D), jnp.float32)

    for d in range(DEPTH):
        parts = _mp_call(tab, cmb3, dst3, zeros_np)
        if d < DEPTH - 1:
            h, tab = _cellt_call(parts, h, cell_Wih, cell_Whh,
                                 cell_bih.reshape(1, 3 * D),
                                 cell_bhh.reshape(1, 3 * D), wtop, ebias)
        else:
            h = _cell_call(parts, h, cell_Wih, cell_Whh,
                           cell_bih.reshape(1, 3 * D),
                           cell_bhh.reshape(1, 3 * D))

    node_post = h
    merged = _post_call(
        node_post, gru_out,
        graph_ids.astype(jnp.int32).reshape(1, N),
        last_graph_node_index.astype(jnp.int32).reshape(B, 1),
        W_post, b_post.reshape(1, 2 * D),
        W_merge, b_merge.reshape(1, D))
    return (node_post, merged)


# speedup vs baseline: 1.1895x; 1.1418x over previous
import functools

import jax
import jax.numpy as jnp
from jax import lax
from jax.experimental import pallas as pl
from jax.experimental.pallas import tpu as pltpu
from jax.experimental.pallas import tpu_sc as plsc

N = 10000
E = 160000
F = 16
D = 128
B = 50
L = 200
NT = 8
ET = 16
DEPTH = 3

NC = 2
NS = 16
NW = NC * NS

NP = 10240
EP = 163840
BP = 56
KE = 128
NCHUNK = EP // (NW * KE)
EW = NCHUNK * KE
ROWS_W = NP // NW
KG = 80
GCHUNK = ROWS_W // KG


def _pre_body(nf_ref, nt_ref, wf_ref, bf_ref, te_ref, ee_ref, wtop_ref,
              wbot_ref, bm_ref, pre_ref, npw_ref, eb_ref):
    dense = jnp.tanh(
        jnp.dot(nf_ref[...], wf_ref[...], preferred_element_type=jnp.float32)
        + bf_ref[...])
    oh = (nt_ref[...] == lax.broadcasted_iota(jnp.int32, (N, NT), 1)
          ).astype(jnp.float32)
    pre = dense + jnp.dot(oh, te_ref[...], preferred_element_type=jnp.float32, precision=lax.Precision.HIGHEST)
    pre_ref[...] = pre
    npw_ref[...] = jnp.dot(pre, wtop_ref[...],
                           preferred_element_type=jnp.float32)
    eb_ref[...] = jnp.dot(ee_ref[...], wbot_ref[...],
                          preferred_element_type=jnp.float32) + bm_ref[...]


def _pre_call(nf, nt2, w_feat, b_feat, type_emb, edge_emb, wtop, wbot, bm):
    return pl.pallas_call(
        _pre_body,
        out_shape=(
            jax.ShapeDtypeStruct((N, D), jnp.float32),
            jax.ShapeDtypeStruct((N, D), jnp.float32),
            jax.ShapeDtypeStruct((ET, D), jnp.float32),
        ),
    )(nf, nt2, w_feat, b_feat, type_emb, edge_emb, wtop, wbot, bm)


def _gru_body(x0_ref, wih_ref, whh_ref, bih_ref, bhh_ref, out_ref,
              seq_ref, h_ref, gi_ref):
    l = pl.program_id(0)
    t = pl.program_id(1)

    @pl.when(t == 0)
    def _():
        h_ref[...] = jnp.zeros_like(h_ref)
        x_all = jnp.where(l == 0, x0_ref[...], seq_ref[...])
        gi_all = jnp.dot(x_all.reshape(L * BP, D), wih_ref[0],
                         preferred_element_type=jnp.float32) + bih_ref[0]
        gi_ref[...] = gi_all.reshape(L, BP, 3 * D)

    gi = gi_ref[pl.ds(t, 1)][0]
    h = h_ref[...]
    gh = jnp.dot(h, whh_ref[0], preferred_element_type=jnp.float32) \
        + bhh_ref[0]
    r = jax.nn.sigmoid(gi[:, :D] + gh[:, :D])
    z = jax.nn.sigmoid(gi[:, D:2 * D] + gh[:, D:2 * D])
    n = jnp.tanh(gi[:, 2 * D:] + r * gh[:, 2 * D:])
    hnew = (1.0 - z) * n + z * h
    h_ref[...] = hnew
    seq_ref[pl.ds(t, 1)] = hnew[None]
    out_ref[0] = hnew


def _gru_call(x0p, wih_all, whh_all, bih_all, bhh_all):
    return pl.pallas_call(
        _gru_body,
        grid=(3, L),
        in_specs=[
            pl.BlockSpec((L, BP, D), lambda l, t: (0, 0, 0)),
            pl.BlockSpec((1, D, 3 * D), lambda l, t: (l, 0, 0)),
            pl.BlockSpec((1, D, 3 * D), lambda l, t: (l, 0, 0)),
            pl.BlockSpec((1, 1, 3 * D), lambda l, t: (l, 0, 0)),
            pl.BlockSpec((1, 1, 3 * D), lambda l, t: (l, 0, 0)),
        ],
        out_specs=pl.BlockSpec((1, BP, D), lambda l, t: (t, 0, 0)),
        out_shape=jax.ShapeDtypeStruct((L, BP, D), jnp.float32),
        scratch_shapes=[
            pltpu.VMEM((L, BP, D), jnp.float32),
            pltpu.VMEM((BP, D), jnp.float32),
            pltpu.VMEM((L, BP, 3 * D), jnp.float32),
        ],
    )(x0p, wih_all, whh_all, bih_all, bhh_all)


def _gather2_body(pre_hbm, npw_hbm, idx_hbm, h0_hbm, hw0_hbm,
                  idx_v, rows_a, rows_b, sem_a, sem_b):
    c = lax.axis_index("c")
    s = lax.axis_index("s")
    w = s * NC + c
    base = w * ROWS_W

    def chunk(i, _):
        off = base + i * KG
        pltpu.sync_copy(idx_hbm.at[pl.ds(off, KG)], idx_v)
        cp_a = pltpu.async_copy(pre_hbm.at[idx_v], rows_a, sem_a)
        cp_b = pltpu.async_copy(npw_hbm.at[idx_v], rows_b, sem_b)
        cp_a.wait()
        cp_b.wait()
        pltpu.sync_copy(rows_a, h0_hbm.at[pl.ds(off, KG)])
        pltpu.sync_copy(rows_b, hw0_hbm.at[pl.ds(off, KG)])
        return 0

    lax.fori_loop(0, GCHUNK, chunk, 0)


def _gather2_call(node_pre, npw, idxp):
    mesh = plsc.VectorSubcoreMesh(core_axis_name="c", subcore_axis_name="s")
    f = pl.kernel(
        _gather2_body,
        out_type=(
            jax.ShapeDtypeStruct((NP, D), jnp.float32),
            jax.ShapeDtypeStruct((NP, D), jnp.float32),
        ),
        mesh=mesh,
        scratch_types=[
            pltpu.VMEM((KG,), jnp.int32),
            pltpu.VMEM((KG, D), jnp.float32),
            pltpu.VMEM((KG, D), jnp.float32),
            pltpu.SemaphoreType.DMA,
            pltpu.SemaphoreType.DMA,
        ],
    )
    return f(node_pre, npw, idxp)


def _mp_body(tab_hbm, cmb_hbm, dst_hbm, zero_hbm, parts_hbm,
             acc_sh, cmb_v, dst_v, rows, sem):
    c = lax.axis_index("c")
    s = lax.axis_index("s")
    w = c * NS + s
    stripe = NP // NS

    pltpu.sync_copy(cmb_hbm.at[w], cmb_v)
    pltpu.sync_copy(dst_hbm.at[w], dst_v)

    pltpu.sync_copy(zero_hbm.at[pl.ds(s * stripe, stripe)],
                    acc_sh.at[pl.ds(s * stripe, stripe)])
    plsc.subcore_barrier()

    def chunk(i, _):
        cp = pltpu.async_copy(tab_hbm.at[cmb_v.at[i]], rows, sem)
        cp.wait()
        pltpu.sync_copy(rows, acc_sh.at[dst_v.at[i]], add=True)
        return 0

    lax.fori_loop(0, NCHUNK, chunk, 0)
    plsc.subcore_barrier()
    pltpu.sync_copy(acc_sh.at[pl.ds(s * stripe, stripe)],
                    parts_hbm.at[c, pl.ds(s * stripe, stripe)])


def _mp_call(tab, cmb3, dst3, zeros_np):
    mesh = plsc.VectorSubcoreMesh(core_axis_name="c", subcore_axis_name="s")
    f = pl.kernel(
        _mp_body,
        out_type=jax.ShapeDtypeStruct((NC, NP, D), jnp.float32),
        mesh=mesh,
        scratch_types=[
            pltpu.VMEM_SHARED((NP, D), jnp.float32),
            pltpu.VMEM((NCHUNK, KE), jnp.int32),
            pltpu.VMEM((NCHUNK, KE), jnp.int32),
            pltpu.VMEM((KE, D), jnp.float32),
            pltpu.SemaphoreType.DMA,
        ],
    )
    return f(tab, cmb3, dst3, zeros_np)


_TRB = 1000


def _table_body(hw_ref, eb_ref, tab_ref):
    t = jnp.tanh(hw_ref[...][:, None, :] + eb_ref[...][None, :, :])
    tab_ref[...] = t.reshape(_TRB * ET, D)


def _table_call(hw, ebias):
    return pl.pallas_call(
        _table_body,
        grid=(N // _TRB,),
        in_specs=[
            pl.BlockSpec((_TRB, D), lambda i: (i, 0)),
            pl.BlockSpec((ET, D), lambda i: (0, 0)),
        ],
        out_specs=pl.BlockSpec((_TRB * ET, D), lambda i: (i, 0)),
        out_shape=jax.ShapeDtypeStruct((N * ET, D), jnp.float32),
    )(hw, ebias)


def _cell_body(p0_ref, p1_ref, h_ref, wih_ref, whh_ref, bih_ref, bhh_ref,
               hn_ref):
    agg = p0_ref[...] + p1_ref[...]
    h = h_ref[...]
    gi = jnp.dot(agg, wih_ref[...], preferred_element_type=jnp.float32) \
        + bih_ref[...]
    gh = jnp.dot(h, whh_ref[...], preferred_element_type=jnp.float32) \
        + bhh_ref[...]
    r = jax.nn.sigmoid(gi[:, :D] + gh[:, :D])
    z = jax.nn.sigmoid(gi[:, D:2 * D] + gh[:, D:2 * D])
    n = jnp.tanh(gi[:, 2 * D:] + r * gh[:, 2 * D:])
    hn_ref[...] = (1.0 - z) * n + z * h


def _cellt_body(p0_ref, p1_ref, h_ref, wih_ref, whh_ref, bih_ref, bhh_ref,
                wtop_ref, eb_ref, hn_ref, tab_ref):
    agg = p0_ref[...] + p1_ref[...]
    h = h_ref[...]
    gi = jnp.dot(agg, wih_ref[...], preferred_element_type=jnp.float32) \
        + bih_ref[...]
    gh = jnp.dot(h, whh_ref[...], preferred_element_type=jnp.float32) \
        + bhh_ref[...]
    r = jax.nn.sigmoid(gi[:, :D] + gh[:, :D])
    z = jax.nn.sigmoid(gi[:, D:2 * D] + gh[:, D:2 * D])
    n = jnp.tanh(gi[:, 2 * D:] + r * gh[:, 2 * D:])
    hnew = (1.0 - z) * n + z * h
    hn_ref[...] = hnew
    hw = jnp.dot(hnew, wtop_ref[...], preferred_element_type=jnp.float32)
    t = jnp.tanh(hw[:, None, :] + eb_ref[...][None, :, :])
    tab_ref[...] = t.reshape(_RB * ET, D)


_RB = 1000


def _cell_call(parts, h, wih, whh, bih, bhh):
    p0 = parts[0, :N]
    p1 = parts[1, :N]
    return pl.pallas_call(
        _cell_body,
        grid=(N // _RB,),
        in_specs=[
            pl.BlockSpec((_RB, D), lambda i: (i, 0)),
            pl.BlockSpec((_RB, D), lambda i: (i, 0)),
            pl.BlockSpec((_RB, D), lambda i: (i, 0)),
            pl.BlockSpec((D, 3 * D), lambda i: (0, 0)),
            pl.BlockSpec((D, 3 * D), lambda i: (0, 0)),
            pl.BlockSpec((1, 3 * D), lambda i: (0, 0)),
            pl.BlockSpec((1, 3 * D), lambda i: (0, 0)),
        ],
        out_specs=pl.BlockSpec((_RB, D), lambda i: (i, 0)),
        out_shape=jax.ShapeDtypeStruct((N, D), jnp.float32),
    )(p0, p1, h, wih, whh, bih, bhh)


def _cellt_call(parts, h, wih, whh, bih, bhh, wtop, ebias):
    p0 = parts[0, :N]
    p1 = parts[1, :N]
    return pl.pallas_call(
        _cellt_body,
        grid=(N // _RB,),
        in_specs=[
            pl.BlockSpec((_RB, D), lambda i: (i, 0)),
            pl.BlockSpec((_RB, D), lambda i: (i, 0)),
            pl.BlockSpec((_RB, D), lambda i: (i, 0)),
            pl.BlockSpec((D, 3 * D), lambda i: (0, 0)),
            pl.BlockSpec((D, 3 * D), lambda i: (0, 0)),
            pl.BlockSpec((1, 3 * D), lambda i: (0, 0)),
            pl.BlockSpec((1, 3 * D), lambda i: (0, 0)),
            pl.BlockSpec((D, D), lambda i: (0, 0)),
            pl.BlockSpec((ET, D), lambda i: (0, 0)),
        ],
        out_specs=(
            pl.BlockSpec((_RB, D), lambda i: (i, 0)),
            pl.BlockSpec((_RB * ET, D), lambda i: (i, 0)),
        ),
        out_shape=(
            jax.ShapeDtypeStruct((N, D), jnp.float32),
            jax.ShapeDtypeStruct((N * ET, D), jnp.float32),
        ),
    )(p0, p1, h, wih, whh, bih, bhh, wtop, ebias)


def _post_body(np_ref, go_ref, gid_ref, lgni_ref, wp_ref, bp_ref,
               wm_ref, bm_ref, out_ref):
    ohg = (lax.broadcasted_iota(jnp.int32, (B, N), 0) == gid_ref[...]
           ).astype(jnp.float32)
    pooled = jnp.dot(ohg, np_ref[...], preferred_element_type=jnp.float32, precision=lax.Precision.HIGHEST)
    gp = jnp.tanh(
        jnp.dot(pooled, wp_ref[...], preferred_element_type=jnp.float32)
        + bp_ref[...])
    ohl = (lax.broadcasted_iota(jnp.int32, (B, N), 1) == lgni_ref[...]
           ).astype(jnp.float32)
    ge = jnp.dot(ohl, go_ref[...], preferred_element_type=jnp.float32, precision=lax.Precision.HIGHEST)
    cat = jnp.concatenate([ge, gp], axis=-1)
    out_ref[...] = (
        jnp.dot(cat, wm_ref[...], preferred_element_type=jnp.float32)
        + bm_ref[...])


def _post_call(node_post, gru_out, gid2, lgni2, w_post, b_post,
               w_merge, b_merge):
    return pl.pallas_call(
        _post_body,
        out_shape=jax.ShapeDtypeStruct((B, D), jnp.float32),
    )(node_post, gru_out, gid2, lgni2, w_post, b_post, w_merge, b_merge)


def kernel(node_features, node_types, last_graph_node_index,
           node_features_graph_index, edge_types, edge_index, graph_ids,
           W_feat, b_feat, type_emb, gru_params, edge_emb, W_msg, b_msg,
           cell_Wih, cell_Whh, cell_bih, cell_bhh, W_post, b_post,
           W_merge, b_merge):
    nt2 = node_types.astype(jnp.int32).reshape(N, 1)
    wtop = W_msg[:D]
    wbot = W_msg[D:]
    bm2 = b_msg.reshape(1, D)

    node_pre, npw, ebias = _pre_call(
        node_features, nt2, W_feat, b_feat.reshape(1, D), type_emb,
        edge_emb, wtop, wbot, bm2)

    x0p = jnp.pad(node_pre.reshape(L, B, D), ((0, 0), (0, BP - B), (0, 0)))
    wih_all = jnp.stack([g[0] for g in gru_params])
    whh_all = jnp.stack([g[1] for g in gru_params])
    bih_all = jnp.stack([g[2] for g in gru_params]).reshape(3, 1, 3 * D)
    bhh_all = jnp.stack([g[3] for g in gru_params]).reshape(3, 1, 3 * D)
    gru_seq = _gru_call(x0p, wih_all, whh_all, bih_all, bhh_all)
    gru_out = gru_seq[:, :B, :].reshape(N, D)

    idxp = jnp.pad(node_features_graph_index.astype(jnp.int32), (0, NP - N))
    h0p, hw0p = _gather2_call(node_pre, npw, idxp)
    h = h0p[:N]
    tab = _table_call(hw0p[:N], ebias)

    srcp = jnp.pad(edge_index[0].astype(jnp.int32), (0, EP - E))
    etp = jnp.pad(edge_types.astype(jnp.int32), (0, EP - E))
    dstp = jnp.pad(edge_index[1].astype(jnp.int32), (0, EP - E),
                   constant_values=N)
    cmb3 = (srcp * ET + etp).reshape(NW, NCHUNK, KE)
    dst3 = dstp.reshape(NW, NCHUNK, KE)
    zeros_np = jnp.zeros((NP, D), jnp.float32)

    for d in range(DEPTH):
        parts = _mp_call(tab, cmb3, dst3, zeros_np)
        if d < DEPTH - 1:
            h, tab = _cellt_call(parts, h, cell_Wih, cell_Whh,
                                 cell_bih.reshape(1, 3 * D),
                                 cell_bhh.reshape(1, 3 * D), wtop, ebias)
        else:
            h = _cell_call(parts, h, cell_Wih, cell_Whh,
                           cell_bih.reshape(1, 3 * D),
                           cell_bhh.reshape(1, 3 * D))

    node_post = h
    merged = _post_call(
        node_post, gru_out,
        graph_ids.astype(jnp.int32).reshape(1, N),
        last_graph_node_index.astype(jnp.int32).reshape(B, 1),
        W_post, b_post.reshape(1, 2 * D),
        W_merge, b_merge.reshape(1, D))
    return (node_post, merged)
